# Initial kernel scaffold; baseline (speedup 1.0000x reference)
import jax
import jax.numpy as jnp
from jax.experimental import pallas as pl


def kernel(inputs, embedding):
    raise NotImplementedError("write your pallas kernel here")


# speedup vs baseline: 2.1126x; 2.1126x over previous
import jax
import jax.numpy as jnp
from jax.experimental import pallas as pl
from jax.experimental.pallas import tpu as pltpu

K = 512
D = 64
B = 8
HW = 576
NGROUP = 8
COMMIT = 0.25


def _vq_kernel(xg_ref, eg_ref, x_ref, emb_ref, q_ref, idx_ref, loss_ref,
               dist_ref, lacc_ref):
    b = pl.program_id(0)
    j = pl.program_id(1)

    xg = xg_ref[0, 0]
    eg = eg_ref[0]

    s = []
    for i in range(8):
        diff = eg[:, i:i + 1] - xg[i:i + 1, :]
        s.append(diff * diff)
    g = (((s[0] + s[4]) + (s[2] + s[6])) + ((s[1] + s[5]) + (s[3] + s[7])))

    @pl.when(j == 0)
    def _():
        dist_ref[...] = g

    @pl.when(j > 0)
    def _():
        dist_ref[...] = dist_ref[...] + g

    @pl.when(j == NGROUP - 1)
    def _():
        dist = dist_ref[...]
        m = jnp.min(dist, axis=0, keepdims=True)
        kiota = jax.lax.broadcasted_iota(jnp.int32, (K, HW), 0)
        idx = jnp.min(jnp.where(dist == m, kiota, K), axis=0)
        idx_ref[0, 0, :] = idx

        onehot = (kiota == idx[None, :]).astype(jnp.float32)
        q = jax.lax.dot_general(
            emb_ref[...], onehot, (((0,), (0,)), ((), ())),
            precision=jax.lax.Precision.HIGHEST,
            preferred_element_type=jnp.float32)
        q_ref[0] = q

        part = jnp.sum((q - x_ref[0]) ** 2)

        @pl.when(b == 0)
        def _():
            lacc_ref[0, 0] = 0.0

        acc = lacc_ref[0, 0] + part
        lacc_ref[0, 0] = acc

        @pl.when(b == B - 1)
        def _():
            loss_ref[0, 0] = (1.0 + COMMIT) * acc / (B * HW * D)


def kernel(inputs, embedding):
    x3 = inputs.reshape(B, D, HW)
    q3, idx3, loss = pl.pallas_call(
        _vq_kernel,
        grid=(B, NGROUP),
        in_specs=[
            pl.BlockSpec((1, 1, 8, HW), lambda b, j: (b, j, 0, 0)),
            pl.BlockSpec((1, K, 8), lambda b, j: (j, 0, 0)),
            pl.BlockSpec((1, D, HW), lambda b, j: (b, 0, 0)),
            pl.BlockSpec((K, D), lambda b, j: (0, 0)),
        ],
        out_specs=[
            pl.BlockSpec((1, D, HW), lambda b, j: (b, 0, 0)),
            pl.BlockSpec((1, 1, HW), lambda b, j: (b, 0, 0)),
            pl.BlockSpec(memory_space=pltpu.SMEM),
        ],
        out_shape=[
            jax.ShapeDtypeStruct((B, D, HW), jnp.float32),
            jax.ShapeDtypeStruct((B, 1, HW), jnp.int32),
            jax.ShapeDtypeStruct((1, 1), jnp.float32),
        ],
        scratch_shapes=[
            pltpu.VMEM((K, HW), jnp.float32),
            pltpu.SMEM((1, 1), jnp.float32),
        ],
    )(x3.reshape(B, NGROUP, 8, HW),
      embedding.reshape(K, NGROUP, 8).transpose(1, 0, 2),
      x3, embedding)
    quantized = q3.reshape(B, D, 24, 24)
    indices = idx3.reshape(B, HW)
    return loss[0, 0], quantized, indices


# speedup vs baseline: 4.0032x; 1.8949x over previous
import jax
import jax.numpy as jnp
from jax.experimental import pallas as pl
from jax.experimental.pallas import tpu as pltpu

K = 512
D = 64
B = 8
HW = 576
T = 5
BIG = 3e38
COMMIT = 0.25


def _exact_dist(ec, x):
    diff = ec - x
    sq = diff * diff
    s = [sq[d:d + 1, :] for d in range(D)]
    dist = None
    for j in range(8):
        t = s[8 * j:8 * j + 8]
        g = ((t[0] + t[4]) + (t[2] + t[6])) + ((t[1] + t[5]) + (t[3] + t[7]))
        dist = g if dist is None else dist + g
    return dist


def _vq_kernel(x_ref, emb_ref, q_ref, idx_ref, loss_ref, lacc_ref):
    b = pl.program_id(0)
    x = x_ref[0]
    emb = emb_ref[...]

    scores = jax.lax.dot_general(
        emb, x, (((1,), (0,)), ((), ())),
        precision=jax.lax.Precision.HIGHEST,
        preferred_element_type=jnp.float32)
    e2 = jnp.sum(emb * emb, axis=1, keepdims=True)
    adj = e2 - (scores + scores)

    kiota = jax.lax.broadcasted_iota(jnp.int32, (K, HW), 0)

    best_dist = None
    best_idx = None
    for t in range(T):
        m = jnp.min(adj, axis=0, keepdims=True)
        idx_t = jnp.min(jnp.where(adj == m, kiota, K), axis=0)
        adj = jnp.where(kiota == idx_t[None, :], BIG, adj)
        onehot = (kiota == idx_t[None, :]).astype(jnp.float32)
        ec = jax.lax.dot_general(
            emb, onehot, (((0,), (0,)), ((), ())),
            precision=jax.lax.Precision.HIGHEST,
            preferred_element_type=jnp.float32)
        dist_t = _exact_dist(ec, x)
        if t == 0:
            best_dist, best_idx, q = dist_t, idx_t, ec
        else:
            better = (dist_t < best_dist) | (
                (dist_t == best_dist) & (idx_t[None, :] < best_idx[None, :]))
            best_dist = jnp.where(better, dist_t, best_dist)
            best_idx = jnp.where(better[0], idx_t, best_idx)
            q = jnp.where(better, ec, q)

    idx_ref[0, 0, :] = best_idx
    q_ref[0] = q

    part = jnp.sum((q - x) ** 2)

    @pl.when(b == 0)
    def _():
        lacc_ref[0, 0] = 0.0

    acc = lacc_ref[0, 0] + part
    lacc_ref[0, 0] = acc

    @pl.when(b == B - 1)
    def _():
        loss_ref[0, 0] = (1.0 + COMMIT) * acc / (B * HW * D)


def kernel(inputs, embedding):
    x3 = inputs.reshape(B, D, HW)
    q3, idx3, loss = pl.pallas_call(
        _vq_kernel,
        grid=(B,),
        in_specs=[
            pl.BlockSpec((1, D, HW), lambda b: (b, 0, 0)),
            pl.BlockSpec((K, D), lambda b: (0, 0)),
        ],
        out_specs=[
            pl.BlockSpec((1, D, HW), lambda b: (b, 0, 0)),
            pl.BlockSpec((1, 1, HW), lambda b: (b, 0, 0)),
            pl.BlockSpec(memory_space=pltpu.SMEM),
        ],
        out_shape=[
            jax.ShapeDtypeStruct((B, D, HW), jnp.float32),
            jax.ShapeDtypeStruct((B, 1, HW), jnp.int32),
            jax.ShapeDtypeStruct((1, 1), jnp.float32),
        ],
        scratch_shapes=[pltpu.SMEM((1, 1), jnp.float32)],
    )(x3, embedding)
    quantized = q3.reshape(B, D, 24, 24)
    indices = idx3.reshape(B, HW)
    return loss[0, 0], quantized, indices


# speedup vs baseline: 5.3377x; 1.3334x over previous
import jax
import jax.numpy as jnp
from jax.experimental import pallas as pl
from jax.experimental.pallas import tpu as pltpu

K = 512
D = 64
B = 8
HW = 576
T = 5
BIG = 3e38
COMMIT = 0.25


def _exact_dist(ec, x):
    diff = ec - x
    sq = diff * diff
    s = [sq[d:d + 1, :] for d in range(D)]
    dist = None
    for j in range(8):
        t = s[8 * j:8 * j + 8]
        g = ((t[0] + t[4]) + (t[2] + t[6])) + ((t[1] + t[5]) + (t[3] + t[7]))
        dist = g if dist is None else dist + g
    return dist


def _vq_kernel(x_ref, emb_ref, eh_ref, em_ref, el_ref, q_ref, idx_ref,
               loss_ref, lacc_ref):
    b = pl.program_id(0)
    x = x_ref[0]
    emb = emb_ref[...]

    scores = jax.lax.dot_general(
        emb, x, (((1,), (0,)), ((), ())),
        precision=jax.lax.Precision.HIGHEST,
        preferred_element_type=jnp.float32)
    e2 = jnp.sum(emb * emb, axis=1, keepdims=True)
    adj = e2 - (scores + scores)

    kiota = jax.lax.broadcasted_iota(jnp.int32, (K, HW), 0)

    best_dist = None
    best_idx = None
    for t in range(T):
        m = jnp.min(adj, axis=0, keepdims=True)
        idx_t = jnp.min(jnp.where(adj == m, kiota, K), axis=0)
        adj = jnp.where(kiota == idx_t[None, :], BIG, adj)
        onehot = (kiota == idx_t[None, :]).astype(jnp.bfloat16)
        parts = []
        for p_ref in (eh_ref, em_ref, el_ref):
            parts.append(jax.lax.dot_general(
                p_ref[...], onehot, (((0,), (0,)), ((), ())),
                preferred_element_type=jnp.float32))
        ec = (parts[0] + parts[1]) + parts[2]
        dist_t = _exact_dist(ec, x)
        if t == 0:
            best_dist, best_idx, q = dist_t, idx_t, ec
        else:
            better = (dist_t < best_dist) | (
                (dist_t == best_dist) & (idx_t[None, :] < best_idx[None, :]))
            best_dist = jnp.where(better, dist_t, best_dist)
            best_idx = jnp.where(better[0], idx_t, best_idx)
            q = jnp.where(better, ec, q)

    idx_ref[0, 0, :] = best_idx
    q_ref[0] = q

    part = jnp.sum((q - x) ** 2)

    @pl.when(b == 0)
    def _():
        lacc_ref[0, 0] = 0.0

    acc = lacc_ref[0, 0] + part
    lacc_ref[0, 0] = acc

    @pl.when(b == B - 1)
    def _():
        loss_ref[0, 0] = (1.0 + COMMIT) * acc / (B * HW * D)


def kernel(inputs, embedding):
    x3 = inputs.reshape(B, D, HW)
    eh = embedding.astype(jnp.bfloat16)
    em = (embedding - eh.astype(jnp.float32)).astype(jnp.bfloat16)
    el = ((embedding - eh.astype(jnp.float32))
          - em.astype(jnp.float32)).astype(jnp.bfloat16)
    q3, idx3, loss = pl.pallas_call(
        _vq_kernel,
        grid=(B,),
        in_specs=[
            pl.BlockSpec((1, D, HW), lambda b: (b, 0, 0)),
            pl.BlockSpec((K, D), lambda b: (0, 0)),
            pl.BlockSpec((K, D), lambda b: (0, 0)),
            pl.BlockSpec((K, D), lambda b: (0, 0)),
            pl.BlockSpec((K, D), lambda b: (0, 0)),
        ],
        out_specs=[
            pl.BlockSpec((1, D, HW), lambda b: (b, 0, 0)),
            pl.BlockSpec((1, 1, HW), lambda b: (b, 0, 0)),
            pl.BlockSpec(memory_space=pltpu.SMEM),
        ],
        out_shape=[
            jax.ShapeDtypeStruct((B, D, HW), jnp.float32),
            jax.ShapeDtypeStruct((B, 1, HW), jnp.int32),
            jax.ShapeDtypeStruct((1, 1), jnp.float32),
        ],
        scratch_shapes=[pltpu.SMEM((1, 1), jnp.float32)],
    )(x3, embedding, eh, em, el)
    quantized = q3.reshape(B, D, 24, 24)
    indices = idx3.reshape(B, HW)
    return loss[0, 0], quantized, indices


# speedup vs baseline: 5.4357x; 1.0184x over previous
import jax
import jax.numpy as jnp
from jax.experimental import pallas as pl
from jax.experimental.pallas import tpu as pltpu

K = 512
D = 64
B = 8
HW = 576
T = 5
BIG = 3e38
COMMIT = 0.25


def _exact_dist(ec, x):
    diff = ec - x
    sq = diff * diff
    s = [sq[d:d + 1, :] for d in range(D)]
    dist = None
    for j in range(8):
        t = s[8 * j:8 * j + 8]
        g = ((t[0] + t[4]) + (t[2] + t[6])) + ((t[1] + t[5]) + (t[3] + t[7]))
        dist = g if dist is None else dist + g
    return dist


def _vq_kernel(x_ref, emb_ref, eh_ref, em_ref, el_ref, q_ref, idx_ref,
               loss_ref, lacc_ref):
    b = pl.program_id(0)
    x = x_ref[0]
    emb = emb_ref[...]

    xh = x.astype(jnp.bfloat16)
    xl = (x - xh.astype(jnp.float32)).astype(jnp.bfloat16)
    eh, em, el = eh_ref[...], em_ref[...], el_ref[...]

    def _dot(a, bb):
        return jax.lax.dot_general(
            a, bb, (((1,), (0,)), ((), ())),
            preferred_element_type=jnp.float32)
    scores = ((_dot(eh, xh) + _dot(eh, xl))
              + (_dot(em, xh) + _dot(el, xh)))
    e2 = jnp.sum(emb * emb, axis=1, keepdims=True)
    adj = e2 - (scores + scores)

    kiota = jax.lax.broadcasted_iota(jnp.int32, (K, HW), 0)

    best_dist = None
    best_idx = None
    for t in range(T):
        m = jnp.min(adj, axis=0, keepdims=True)
        idx_t = jnp.min(jnp.where(adj == m, kiota, K), axis=0)
        adj = jnp.where(kiota == idx_t[None, :], BIG, adj)
        onehot = (kiota == idx_t[None, :]).astype(jnp.bfloat16)
        parts = []
        for p in (eh, em, el):
            parts.append(jax.lax.dot_general(
                p, onehot, (((0,), (0,)), ((), ())),
                preferred_element_type=jnp.float32))
        ec = (parts[0] + parts[1]) + parts[2]
        dist_t = _exact_dist(ec, x)
        if t == 0:
            best_dist, best_idx, q = dist_t, idx_t, ec
        else:
            better = (dist_t < best_dist) | (
                (dist_t == best_dist) & (idx_t[None, :] < best_idx[None, :]))
            best_dist = jnp.where(better, dist_t, best_dist)
            best_idx = jnp.where(better[0], idx_t, best_idx)
            q = jnp.where(better, ec, q)

    idx_ref[0, 0, :] = best_idx
    q_ref[0] = q

    part = jnp.sum((q - x) ** 2)

    @pl.when(b == 0)
    def _():
        lacc_ref[0, 0] = 0.0

    acc = lacc_ref[0, 0] + part
    lacc_ref[0, 0] = acc

    @pl.when(b == B - 1)
    def _():
        loss_ref[0, 0] = (1.0 + COMMIT) * acc / (B * HW * D)


def kernel(inputs, embedding):
    x3 = inputs.reshape(B, D, HW)
    eh = embedding.astype(jnp.bfloat16)
    em = (embedding - eh.astype(jnp.float32)).astype(jnp.bfloat16)
    el = ((embedding - eh.astype(jnp.float32))
          - em.astype(jnp.float32)).astype(jnp.bfloat16)
    q3, idx3, loss = pl.pallas_call(
        _vq_kernel,
        grid=(B,),
        in_specs=[
            pl.BlockSpec((1, D, HW), lambda b: (b, 0, 0)),
            pl.BlockSpec((K, D), lambda b: (0, 0)),
            pl.BlockSpec((K, D), lambda b: (0, 0)),
            pl.BlockSpec((K, D), lambda b: (0, 0)),
            pl.BlockSpec((K, D), lambda b: (0, 0)),
        ],
        out_specs=[
            pl.BlockSpec((1, D, HW), lambda b: (b, 0, 0)),
            pl.BlockSpec((1, 1, HW), lambda b: (b, 0, 0)),
            pl.BlockSpec(memory_space=pltpu.SMEM),
        ],
        out_shape=[
            jax.ShapeDtypeStruct((B, D, HW), jnp.float32),
            jax.ShapeDtypeStruct((B, 1, HW), jnp.int32),
            jax.ShapeDtypeStruct((1, 1), jnp.float32),
        ],
        scratch_shapes=[pltpu.SMEM((1, 1), jnp.float32)],
    )(x3, embedding, eh, em, el)
    quantized = q3.reshape(B, D, 24, 24)
    indices = idx3.reshape(B, HW)
    return loss[0, 0], quantized, indices


# speedup vs baseline: 5.4670x; 1.0058x over previous
import jax
import jax.numpy as jnp
from jax.experimental import pallas as pl
from jax.experimental.pallas import tpu as pltpu

K = 512
D = 64
B = 8
HW = 576
T = 5
BIG = 3e38
COMMIT = 0.25


def _exact_dist(ec, x):
    diff = ec - x
    sq = diff * diff
    s = [sq[d:d + 1, :] for d in range(D)]
    dist = None
    for j in range(8):
        t = s[8 * j:8 * j + 8]
        g = ((t[0] + t[4]) + (t[2] + t[6])) + ((t[1] + t[5]) + (t[3] + t[7]))
        dist = g if dist is None else dist + g
    return dist


def _vq_kernel(x_ref, emb_ref, eh_ref, em_ref, el_ref, q_ref, idx_ref,
               loss_ref, lacc_ref):
    b = pl.program_id(0)
    x = x_ref[0]
    emb = emb_ref[...]

    xh = x.astype(jnp.bfloat16)
    xl = (x - xh.astype(jnp.float32)).astype(jnp.bfloat16)
    eh, em, el = eh_ref[...], em_ref[...], el_ref[...]

    def _dot(a, bb):
        return jax.lax.dot_general(
            a, bb, (((1,), (0,)), ((), ())),
            preferred_element_type=jnp.float32)
    scores = ((_dot(eh, xh) + _dot(eh, xl))
              + (_dot(em, xh) + _dot(el, xh)))
    e2 = jnp.sum(emb * emb, axis=1, keepdims=True)
    adj = e2 - (scores + scores)

    kiota = jax.lax.broadcasted_iota(jnp.int32, (K, HW), 0)

    best_dist = None
    best_idx = None
    for t in range(T):
        m = jnp.min(adj, axis=0, keepdims=True)
        idx_t = jnp.min(jnp.where(adj == m, kiota, K), axis=0)
        ohb = kiota == idx_t[None, :]
        adj = jnp.where(ohb, BIG, adj)
        onehot = ohb.astype(jnp.bfloat16)
        parts = []
        for p in (eh, em, el):
            parts.append(jax.lax.dot_general(
                p, onehot, (((0,), (0,)), ((), ())),
                preferred_element_type=jnp.float32))
        ec = (parts[0] + parts[1]) + parts[2]
        dist_t = _exact_dist(ec, x)
        if t == 0:
            best_dist, best_idx, q = dist_t, idx_t, ec
        else:
            better = (dist_t < best_dist) | (
                (dist_t == best_dist) & (idx_t[None, :] < best_idx[None, :]))
            best_dist = jnp.where(better, dist_t, best_dist)
            best_idx = jnp.where(better[0], idx_t, best_idx)
            q = jnp.where(better, ec, q)

    idx_ref[0, 0, :] = best_idx
    q_ref[0] = q

    part = jnp.sum((q - x) ** 2)

    @pl.when(b == 0)
    def _():
        lacc_ref[0, 0] = 0.0

    acc = lacc_ref[0, 0] + part
    lacc_ref[0, 0] = acc

    @pl.when(b == B - 1)
    def _():
        loss_ref[0, 0] = (1.0 + COMMIT) * acc / (B * HW * D)


def kernel(inputs, embedding):
    x3 = inputs.reshape(B, D, HW)
    eh = embedding.astype(jnp.bfloat16)
    em = (embedding - eh.astype(jnp.float32)).astype(jnp.bfloat16)
    el = ((embedding - eh.astype(jnp.float32))
          - em.astype(jnp.float32)).astype(jnp.bfloat16)
    q3, idx3, loss = pl.pallas_call(
        _vq_kernel,
        grid=(B,),
        in_specs=[
            pl.BlockSpec((1, D, HW), lambda b: (b, 0, 0)),
            pl.BlockSpec((K, D), lambda b: (0, 0)),
            pl.BlockSpec((K, D), lambda b: (0, 0)),
            pl.BlockSpec((K, D), lambda b: (0, 0)),
            pl.BlockSpec((K, D), lambda b: (0, 0)),
        ],
        out_specs=[
            pl.BlockSpec((1, D, HW), lambda b: (b, 0, 0)),
            pl.BlockSpec((1, 1, HW), lambda b: (b, 0, 0)),
            pl.BlockSpec(memory_space=pltpu.SMEM),
        ],
        out_shape=[
            jax.ShapeDtypeStruct((B, D, HW), jnp.float32),
            jax.ShapeDtypeStruct((B, 1, HW), jnp.int32),
            jax.ShapeDtypeStruct((1, 1), jnp.float32),
        ],
        scratch_shapes=[pltpu.SMEM((1, 1), jnp.float32)],
    )(x3, embedding, eh, em, el)
    quantized = q3.reshape(B, D, 24, 24)
    indices = idx3.reshape(B, HW)
    return loss[0, 0], quantized, indices


# speedup vs baseline: 5.6946x; 1.0416x over previous
import jax
import jax.numpy as jnp
from jax.experimental import pallas as pl
from jax.experimental.pallas import tpu as pltpu

K = 512
D = 64
B = 8
HW = 576
T = 5
BIG = 3e38
COMMIT = 0.25


def _exact_dist(ec, x):
    diff = ec - x
    sq = diff * diff
    s = [sq[d:d + 1, :] for d in range(D)]
    dist = None
    for j in range(8):
        t = s[8 * j:8 * j + 8]
        g = ((t[0] + t[4]) + (t[2] + t[6])) + ((t[1] + t[5]) + (t[3] + t[7]))
        dist = g if dist is None else dist + g
    return dist


NB = 2


def _vq_kernel(x_ref, emb_ref, eh_ref, em_ref, el_ref, q_ref, idx_ref,
               loss_ref, lacc_ref):
    b = pl.program_id(0)
    emb = emb_ref[...]
    eh, em, el = eh_ref[...], em_ref[...], el_ref[...]
    e2 = jnp.sum(emb * emb, axis=1, keepdims=True)
    kiota = jax.lax.broadcasted_iota(jnp.int32, (K, HW), 0)

    def _dot(a, bb):
        return jax.lax.dot_general(
            a, bb, (((1,), (0,)), ((), ())),
            preferred_element_type=jnp.float32)

    part = 0.0
    for c in range(NB):
        x = x_ref[c]
        xh = x.astype(jnp.bfloat16)
        xl = (x - xh.astype(jnp.float32)).astype(jnp.bfloat16)
        scores = ((_dot(eh, xh) + _dot(eh, xl))
                  + (_dot(em, xh) + _dot(el, xh)))
        adj = e2 - (scores + scores)

        best_dist = None
        best_idx = None
        for t in range(T):
            m = jnp.min(adj, axis=0, keepdims=True)
            idx_t = jnp.min(jnp.where(adj == m, kiota, K), axis=0)
            ohb = kiota == idx_t[None, :]
            adj = jnp.where(ohb, BIG, adj)
            onehot = ohb.astype(jnp.bfloat16)
            parts = []
            for p in (eh, em, el):
                parts.append(jax.lax.dot_general(
                    p, onehot, (((0,), (0,)), ((), ())),
                    preferred_element_type=jnp.float32))
            ec = (parts[0] + parts[1]) + parts[2]
            dist_t = _exact_dist(ec, x)
            if t == 0:
                best_dist, best_idx, q = dist_t, idx_t, ec
            else:
                better = (dist_t < best_dist) | (
                    (dist_t == best_dist)
                    & (idx_t[None, :] < best_idx[None, :]))
                best_dist = jnp.where(better, dist_t, best_dist)
                best_idx = jnp.where(better[0], idx_t, best_idx)
                q = jnp.where(better, ec, q)

        idx_ref[c, 0, :] = best_idx
        q_ref[c] = q
        part = part + jnp.sum((q - x) ** 2)

    @pl.when(b == 0)
    def _():
        lacc_ref[0, 0] = 0.0

    acc = lacc_ref[0, 0] + part
    lacc_ref[0, 0] = acc

    @pl.when(b == B // NB - 1)
    def _():
        loss_ref[0, 0] = (1.0 + COMMIT) * acc / (B * HW * D)


def kernel(inputs, embedding):
    x3 = inputs.reshape(B, D, HW)
    eh = embedding.astype(jnp.bfloat16)
    em = (embedding - eh.astype(jnp.float32)).astype(jnp.bfloat16)
    el = ((embedding - eh.astype(jnp.float32))
          - em.astype(jnp.float32)).astype(jnp.bfloat16)
    q3, idx3, loss = pl.pallas_call(
        _vq_kernel,
        grid=(B // NB,),
        in_specs=[
            pl.BlockSpec((NB, D, HW), lambda b: (b, 0, 0)),
            pl.BlockSpec((K, D), lambda b: (0, 0)),
            pl.BlockSpec((K, D), lambda b: (0, 0)),
            pl.BlockSpec((K, D), lambda b: (0, 0)),
            pl.BlockSpec((K, D), lambda b: (0, 0)),
        ],
        out_specs=[
            pl.BlockSpec((NB, D, HW), lambda b: (b, 0, 0)),
            pl.BlockSpec((NB, 1, HW), lambda b: (b, 0, 0)),
            pl.BlockSpec(memory_space=pltpu.SMEM),
        ],
        out_shape=[
            jax.ShapeDtypeStruct((B, D, HW), jnp.float32),
            jax.ShapeDtypeStruct((B, 1, HW), jnp.int32),
            jax.ShapeDtypeStruct((1, 1), jnp.float32),
        ],
        scratch_shapes=[pltpu.SMEM((1, 1), jnp.float32)],
    )(x3, embedding, eh, em, el)
    quantized = q3.reshape(B, D, 24, 24)
    indices = idx3.reshape(B, HW)
    return loss[0, 0], quantized, indices
